# baseline (speedup 1.0000x reference)
import jax
import jax.numpy as jnp
from jax import lax
from jax.experimental import pallas as pl
from jax.experimental.pallas import tpu as pltpu
from jax.experimental.pallas import tpu_sc as plsc

N = 10000
E = 320000
NFEAT = 128
NHID = 128
NCLASS = 64

NC = 2
NS = 16
NW = NC * NS
N_PAD = 10240
EPW = E // NW
CHUNK = 80
NCHUNK = EPW // CHUNK
D = 64


ROWS = 1000


def _mm1_body(x_ref, w_ref, o0_ref, oa_ref, ob_ref):
    r = lax.dot_general(
        x_ref[...], w_ref[...], (((1,), (1,)), ((), ())),
        preferred_element_type=jnp.float32)
    o0_ref[...] = r[:, :NHID]
    oa_ref[...] = r[:, NHID:NHID + D]
    ob_ref[...] = r[:, NHID + D:]


def _mm1(x, wcat):
    n, k = x.shape
    return pl.pallas_call(
        _mm1_body,
        grid=(n // ROWS,),
        in_specs=[pl.BlockSpec((ROWS, k), lambda i: (i, 0)),
                  pl.BlockSpec((2 * NHID, k), lambda i: (0, 0))],
        out_specs=[pl.BlockSpec((ROWS, NHID), lambda i: (i, 0)),
                   pl.BlockSpec((ROWS, D), lambda i: (i, 0)),
                   pl.BlockSpec((ROWS, D), lambda i: (i, 0))],
        out_shape=[jax.ShapeDtypeStruct((n, NHID), jnp.float32),
                   jax.ShapeDtypeStruct((n, D), jnp.float32),
                   jax.ShapeDtypeStruct((n, D), jnp.float32)],
    )(x, wcat)


def _combine_body(xw0_ref, pa0_ref, pa1_ref, pb0_ref, pb1_ref, b_ref, w_ref,
                  o0_ref, o1_ref):
    p = jnp.concatenate([pa0_ref[...] + pa1_ref[...],
                         pb0_ref[...] + pb1_ref[...]], axis=1)
    h = jnp.maximum(xw0_ref[...] + p + b_ref[...], 0.0)
    r = lax.dot_general(
        h, w_ref[...], (((1,), (1,)), ((), ())),
        preferred_element_type=jnp.float32)
    o0_ref[...] = r[:, :NCLASS]
    o1_ref[...] = r[:, NCLASS:]


def _combine1(xw0, pa0, pa1, pb0, pb1, b, wcat2):
    n = xw0.shape[0]
    return pl.pallas_call(
        _combine_body,
        grid=(n // ROWS,),
        in_specs=[pl.BlockSpec((ROWS, NHID), lambda i: (i, 0)),
                  pl.BlockSpec((ROWS, D), lambda i: (i, 0)),
                  pl.BlockSpec((ROWS, D), lambda i: (i, 0)),
                  pl.BlockSpec((ROWS, D), lambda i: (i, 0)),
                  pl.BlockSpec((ROWS, D), lambda i: (i, 0)),
                  pl.BlockSpec((1, NHID), lambda i: (0, 0)),
                  pl.BlockSpec((2 * NCLASS, NHID), lambda i: (0, 0))],
        out_specs=[pl.BlockSpec((ROWS, NCLASS), lambda i: (i, 0)),
                   pl.BlockSpec((ROWS, NCLASS), lambda i: (i, 0))],
        out_shape=[jax.ShapeDtypeStruct((n, NCLASS), jnp.float32),
                   jax.ShapeDtypeStruct((n, NCLASS), jnp.float32)],
    )(xw0, pa0, pa1, pb0, pb1, b, wcat2)


def _final_body(hw0_ref, q0_ref, q1_ref, b_ref, o_ref):
    z = hw0_ref[...] + q0_ref[...] + q1_ref[...] + b_ref[...]
    m = jnp.max(z, axis=1, keepdims=True)
    zs = z - m
    lse = jnp.log(jnp.sum(jnp.exp(zs), axis=1, keepdims=True))
    o_ref[...] = zs - lse


def _final(hw0, q0, q1, b):
    n, k = hw0.shape
    return pl.pallas_call(
        _final_body,
        grid=(n // ROWS,),
        in_specs=[pl.BlockSpec((ROWS, k), lambda i: (i, 0)),
                  pl.BlockSpec((ROWS, k), lambda i: (i, 0)),
                  pl.BlockSpec((ROWS, k), lambda i: (i, 0)),
                  pl.BlockSpec((1, k), lambda i: (0, 0))],
        out_specs=pl.BlockSpec((ROWS, k), lambda i: (i, 0)),
        out_shape=jax.ShapeDtypeStruct((n, k), jnp.float32),
    )(hw0, q0, q1, b)


_SC_MESH = plsc.VectorSubcoreMesh(core_axis_name="c", subcore_axis_name="s")
_SC_PARAMS = pltpu.CompilerParams(needs_layout_passes=False,
                                  use_tc_tiling_on_sc=False)


def _rsqrt16(v):
    xi = plsc.bitcast(v, jnp.int32)
    xi = jnp.int32(0x5F3759DF) - lax.shift_right_arithmetic(xi, 1)
    y = plsc.bitcast(xi, jnp.float32)
    half = v * 0.5
    for _ in range(3):
        y = y * (1.5 - half * y * y)
    return y


NBUF = 4


def _agg_passes(ys, out_hbm, row_v, col_v, nrm_v, rows_v, agg_sh,
                gsems, ssems, sid, cid):
    nsl = N_PAD // NS

    def fire_gather(y_hbm, j, b):
        pltpu.async_copy(y_hbm.at[col_v.at[j]], rows_v.at[b], gsems[b])

    def wait_gather(y_hbm, j, b):
        pltpu.make_async_copy(y_hbm.at[col_v.at[j]], rows_v.at[b],
                              gsems[b]).wait()

    def fire_scatter(j, b):
        pltpu.async_copy(rows_v.at[b], agg_sh.at[row_v.at[j]], ssems[b],
                         add=True)

    def wait_scatter(j, b):
        pltpu.make_async_copy(rows_v.at[b], agg_sh.at[row_v.at[j]],
                              ssems[b]).wait()

    def scale(j, b):
        def scale_step(i, _):
            for u in range(4):
                e = i * 4 + u
                nb = plsc.load_gather(nrm_v.at[j],
                                      [jnp.full((16,), e, jnp.int32)])
                for dd in range(D // 16):
                    off = pl.multiple_of(dd * 16, 16)
                    rows_v[b, e, pl.ds(off, 16)] = (
                        rows_v[b, e, pl.ds(off, 16)] * nb)
            return 0
        lax.fori_loop(0, CHUNK // 4, scale_step, 0)

    def run_pass(y_hbm, p):
        def zero_row(e, _):
            for dd in range(D // 16):
                rows_v[0, e, pl.ds(dd * 16, 16)] = jnp.zeros(
                    (16,), jnp.float32)
            return 0
        lax.fori_loop(0, CHUNK, zero_row, 0)

        def zero_sh(i, _):
            pltpu.sync_copy(rows_v.at[0],
                            agg_sh.at[pl.ds(sid * nsl + i * CHUNK, CHUNK)])
            return 0
        lax.fori_loop(0, nsl // CHUNK, zero_sh, 0)
        plsc.subcore_barrier()

        def body(j, b, b2):
            @pl.when(j >= 2)
            def _():
                wait_scatter(j - 2, b2)

            @pl.when(j + 2 < NCHUNK)
            def _():
                fire_gather(y_hbm, j + 2, b2)

            wait_gather(y_hbm, j, b)
            scale(j, b)
            fire_scatter(j, b)

        fire_gather(y_hbm, 0, 0)
        fire_gather(y_hbm, 1, 1)

        def group(q, _):
            for u in range(NBUF):
                j = q * NBUF + u
                body(j, u, (u + 2) % NBUF)
            return 0
        ngroup = NCHUNK // NBUF
        lax.fori_loop(0, ngroup, group, 0)
        for u in range(NCHUNK % NBUF):
            j = ngroup * NBUF + u
            body(j, j % NBUF, (j + 2) % NBUF)

        wait_scatter(NCHUNK - 2, (NCHUNK - 2) % NBUF)
        wait_scatter(NCHUNK - 1, (NCHUNK - 1) % NBUF)

        plsc.subcore_barrier()
        pltpu.sync_copy(agg_sh.at[pl.ds(sid * nsl, nsl)],
                        out_hbm.at[p, cid, pl.ds(sid * nsl, nsl)])

    for p, y_hbm in enumerate(ys):
        run_pass(y_hbm, p)


def _l1_body(roww_hbm, colw_hbm, ww_hbm, ya_hbm, yb_hbm, norm_hbm, pp_hbm,
             row_v, col_v, w_v, deg_v, dis_v, nrm_v, rows_v,
             deg_sh, dis_sh, agg_sh, gsems, ssems):
    cid = lax.axis_index("c")
    sid = lax.axis_index("s")
    wid = sid * NC + cid
    nsl = N_PAD // NS

    def zero_dis(i, _):
        off = pl.multiple_of(i * 16, 16)
        dis_v[pl.ds(off, 16)] = jnp.zeros((16,), jnp.float32)
        return 0
    lax.fori_loop(0, nsl // 16, zero_dis, 0)
    pltpu.sync_copy(dis_v, deg_sh.at[pl.ds(sid * nsl, nsl)])
    plsc.subcore_barrier()

    for c2 in range(2):
        slot = sid * 2 + c2
        pltpu.sync_copy(roww_hbm.at[slot], row_v)
        pltpu.sync_copy(ww_hbm.at[slot], w_v)

        def deg_step(j, _):
            pltpu.sync_copy(w_v.at[j], deg_sh.at[row_v.at[j]], add=True)
            return 0
        lax.fori_loop(0, NCHUNK, deg_step, 0)
    plsc.subcore_barrier()

    pltpu.sync_copy(deg_sh.at[pl.ds(sid * nsl, nsl)], dis_v)

    def dis_step(i, _):
        off = pl.multiple_of(i * 16, 16)
        acc = dis_v[pl.ds(off, 16)]
        r = _rsqrt16(acc)
        r = jnp.where(acc > 0.0, r, 0.0)
        dis_v[pl.ds(off, 16)] = r
        return 0
    lax.fori_loop(0, nsl // 16, dis_step, 0)

    pltpu.sync_copy(dis_v, dis_sh.at[pl.ds(sid * nsl, nsl)])
    plsc.subcore_barrier()

    pltpu.sync_copy(dis_sh, deg_v)
    pltpu.sync_copy(roww_hbm.at[wid], row_v)
    pltpu.sync_copy(colw_hbm.at[wid], col_v)
    pltpu.sync_copy(ww_hbm.at[wid], w_v)

    def nrm_step(j, _):
        for dd in range(CHUNK // 16):
            off = pl.multiple_of(dd * 16, 16)
            r = row_v[j, pl.ds(off, 16)]
            c = col_v[j, pl.ds(off, 16)]
            wv = w_v[j, pl.ds(off, 16)]
            dr = plsc.load_gather(deg_v, [r])
            dc = plsc.load_gather(deg_v, [c])
            nrm_v[j, pl.ds(off, 16)] = -(dr * wv * dc)
        return 0
    lax.fori_loop(0, NCHUNK, nrm_step, 0)

    pltpu.sync_copy(nrm_v, norm_hbm.at[wid])

    _agg_passes((ya_hbm, yb_hbm), pp_hbm, row_v, col_v, nrm_v, rows_v,
                agg_sh, gsems, ssems, sid, cid)


def _l1_call(roww, colw, ww, ya, yb):
    kfn = pl.kernel(
        _l1_body,
        out_type=[
            jax.ShapeDtypeStruct((NW, NCHUNK, CHUNK), jnp.float32),
            jax.ShapeDtypeStruct((2, NC, N_PAD, D), jnp.float32),
        ],
        mesh=_SC_MESH,
        compiler_params=_SC_PARAMS,
        scratch_types=[
            pltpu.VMEM((NCHUNK, CHUNK), jnp.int32),
            pltpu.VMEM((NCHUNK, CHUNK), jnp.int32),
            pltpu.VMEM((NCHUNK, CHUNK), jnp.float32),
            pltpu.VMEM((N_PAD,), jnp.float32),
            pltpu.VMEM((N_PAD // NS,), jnp.float32),
            pltpu.VMEM((NCHUNK, CHUNK), jnp.float32),
            pltpu.VMEM((NBUF, CHUNK, D), jnp.float32),
            pltpu.VMEM_SHARED((N_PAD,), jnp.float32),
            pltpu.VMEM_SHARED((N_PAD,), jnp.float32),
            pltpu.VMEM_SHARED((N_PAD, D), jnp.float32),
            [pltpu.SemaphoreType.DMA] * NBUF,
            [pltpu.SemaphoreType.DMA] * NBUF,
        ],
    )
    return kfn(roww, colw, ww, ya, yb)


def _l2_body(y_hbm, row_hbm, col_hbm, nrm_hbm, out_hbm,
             row_v, col_v, nrm_v, rows_v, agg_sh, gsems, ssems):
    cid = lax.axis_index("c")
    sid = lax.axis_index("s")
    wid = sid * NC + cid

    pltpu.sync_copy(row_hbm.at[wid], row_v)
    pltpu.sync_copy(col_hbm.at[wid], col_v)
    pltpu.sync_copy(nrm_hbm.at[wid], nrm_v)

    _agg_passes((y_hbm,), out_hbm, row_v, col_v, nrm_v, rows_v,
                agg_sh, gsems, ssems, sid, cid)


def _l2_call(y, row2, col2, nrm2):
    kfn = pl.kernel(
        _l2_body,
        out_type=jax.ShapeDtypeStruct((1, NC, N_PAD, D), jnp.float32),
        mesh=_SC_MESH,
        compiler_params=_SC_PARAMS,
        scratch_types=[
            pltpu.VMEM((NCHUNK, CHUNK), jnp.int32),
            pltpu.VMEM((NCHUNK, CHUNK), jnp.int32),
            pltpu.VMEM((NCHUNK, CHUNK), jnp.float32),
            pltpu.VMEM((NBUF, CHUNK, D), jnp.float32),
            pltpu.VMEM_SHARED((N_PAD, D), jnp.float32),
            [pltpu.SemaphoreType.DMA] * NBUF,
            [pltpu.SemaphoreType.DMA] * NBUF,
        ],
    )
    return kfn(y, row2, col2, nrm2)


def kernel(features, edge_index, edge_weight, W0_1, W1_1, b1, W0_2, W1_2, b2):
    row = edge_index[0]
    col = edge_index[1]
    row2 = row.reshape(NW, NCHUNK, CHUNK)
    col2 = col.reshape(NW, NCHUNK, CHUNK)

    wcat1 = jnp.concatenate([W0_1, W1_1], axis=0)
    xw0, y1a, y1b = _mm1(features, wcat1)

    nrm2, pp = _l1_call(row2, col2, edge_weight.reshape(NW, NCHUNK, CHUNK),
                        y1a, y1b)

    wcat2 = jnp.concatenate([W0_2, W1_2], axis=0)
    hw0, y2 = _combine1(xw0, pp[0, 0, :N], pp[0, 1, :N],
                        pp[1, 0, :N], pp[1, 1, :N],
                        b1.reshape(1, NHID), wcat2)

    qq = _l2_call(y2, row2, col2, nrm2)
    q = qq[0]

    return _final(hw0, q[0, :N], q[1, :N], b2.reshape(1, NCLASS))


# speedup vs baseline: 1.0251x; 1.0251x over previous
import jax
import jax.numpy as jnp
from jax import lax
from jax.experimental import pallas as pl
from jax.experimental.pallas import tpu as pltpu
from jax.experimental.pallas import tpu_sc as plsc

N = 10000
E = 320000
NFEAT = 128
NHID = 128
NCLASS = 64

NC = 2
NS = 16
NW = NC * NS
N_PAD = 10240
EPW = E // NW
CHUNK = 80
NCHUNK = EPW // CHUNK
D = 64


ROWS = 1000


def _mm1_body(x_ref, w_ref, o0_ref, oa_ref, ob_ref):
    r = lax.dot_general(
        x_ref[...], w_ref[...], (((1,), (1,)), ((), ())),
        preferred_element_type=jnp.float32)
    o0_ref[...] = r[:, :NHID]
    oa_ref[...] = r[:, NHID:NHID + D]
    ob_ref[...] = r[:, NHID + D:]


def _mm1(x, wcat):
    n, k = x.shape
    return pl.pallas_call(
        _mm1_body,
        grid=(n // ROWS,),
        in_specs=[pl.BlockSpec((ROWS, k), lambda i: (i, 0)),
                  pl.BlockSpec((2 * NHID, k), lambda i: (0, 0))],
        out_specs=[pl.BlockSpec((ROWS, NHID), lambda i: (i, 0)),
                   pl.BlockSpec((ROWS, D), lambda i: (i, 0)),
                   pl.BlockSpec((ROWS, D), lambda i: (i, 0))],
        out_shape=[jax.ShapeDtypeStruct((n, NHID), jnp.float32),
                   jax.ShapeDtypeStruct((n, D), jnp.float32),
                   jax.ShapeDtypeStruct((n, D), jnp.float32)],
    )(x, wcat)


def _combine_body(xw0_ref, pa0_ref, pa1_ref, pb0_ref, pb1_ref, b_ref, w_ref,
                  o0_ref, o1_ref):
    p = jnp.concatenate([pa0_ref[...] + pa1_ref[...],
                         pb0_ref[...] + pb1_ref[...]], axis=1)
    h = jnp.maximum(xw0_ref[...] + p + b_ref[...], 0.0)
    r = lax.dot_general(
        h, w_ref[...], (((1,), (1,)), ((), ())),
        preferred_element_type=jnp.float32)
    o0_ref[...] = r[:, :NCLASS]
    o1_ref[...] = r[:, NCLASS:]


def _combine1(xw0, pa0, pa1, pb0, pb1, b, wcat2):
    n = xw0.shape[0]
    return pl.pallas_call(
        _combine_body,
        grid=(n // ROWS,),
        in_specs=[pl.BlockSpec((ROWS, NHID), lambda i: (i, 0)),
                  pl.BlockSpec((ROWS, D), lambda i: (i, 0)),
                  pl.BlockSpec((ROWS, D), lambda i: (i, 0)),
                  pl.BlockSpec((ROWS, D), lambda i: (i, 0)),
                  pl.BlockSpec((ROWS, D), lambda i: (i, 0)),
                  pl.BlockSpec((1, NHID), lambda i: (0, 0)),
                  pl.BlockSpec((2 * NCLASS, NHID), lambda i: (0, 0))],
        out_specs=[pl.BlockSpec((ROWS, NCLASS), lambda i: (i, 0)),
                   pl.BlockSpec((ROWS, NCLASS), lambda i: (i, 0))],
        out_shape=[jax.ShapeDtypeStruct((n, NCLASS), jnp.float32),
                   jax.ShapeDtypeStruct((n, NCLASS), jnp.float32)],
    )(xw0, pa0, pa1, pb0, pb1, b, wcat2)


def _final_body(hw0_ref, q0_ref, q1_ref, b_ref, o_ref):
    z = hw0_ref[...] + q0_ref[...] + q1_ref[...] + b_ref[...]
    m = jnp.max(z, axis=1, keepdims=True)
    zs = z - m
    lse = jnp.log(jnp.sum(jnp.exp(zs), axis=1, keepdims=True))
    o_ref[...] = zs - lse


def _final(hw0, q0, q1, b):
    n, k = hw0.shape
    return pl.pallas_call(
        _final_body,
        grid=(n // ROWS,),
        in_specs=[pl.BlockSpec((ROWS, k), lambda i: (i, 0)),
                  pl.BlockSpec((ROWS, k), lambda i: (i, 0)),
                  pl.BlockSpec((ROWS, k), lambda i: (i, 0)),
                  pl.BlockSpec((1, k), lambda i: (0, 0))],
        out_specs=pl.BlockSpec((ROWS, k), lambda i: (i, 0)),
        out_shape=jax.ShapeDtypeStruct((n, k), jnp.float32),
    )(hw0, q0, q1, b)


_SC_MESH = plsc.VectorSubcoreMesh(core_axis_name="c", subcore_axis_name="s")
_SC_PARAMS = pltpu.CompilerParams(needs_layout_passes=False,
                                  use_tc_tiling_on_sc=False)


def _rsqrt16(v):
    xi = plsc.bitcast(v, jnp.int32)
    xi = jnp.int32(0x5F3759DF) - lax.shift_right_arithmetic(xi, 1)
    y = plsc.bitcast(xi, jnp.float32)
    half = v * 0.5
    for _ in range(3):
        y = y * (1.5 - half * y * y)
    return y


NBUF = 4


def _agg_passes(ys, out_hbm, row_v, col_v, nrm_v, rows_v, agg_sh,
                gsems, ssems, sid, cid):
    nsl = N_PAD // NS

    def fire_gather(y_hbm, j, b):
        pltpu.async_copy(y_hbm.at[col_v.at[j]], rows_v.at[b], gsems[b])

    def wait_gather(y_hbm, j, b):
        pltpu.make_async_copy(y_hbm.at[col_v.at[j]], rows_v.at[b],
                              gsems[b]).wait()

    def fire_scatter(j, b):
        pltpu.async_copy(rows_v.at[b], agg_sh.at[row_v.at[j]], ssems[b],
                         add=True)

    def wait_scatter(j, b):
        pltpu.make_async_copy(rows_v.at[b], agg_sh.at[row_v.at[j]],
                              ssems[b]).wait()

    def scale(j, b):
        def scale_step(i, _):
            for u in range(4):
                e = i * 4 + u
                nb = plsc.load_gather(nrm_v.at[j],
                                      [jnp.full((16,), e, jnp.int32)])
                for dd in range(D // 16):
                    off = pl.multiple_of(dd * 16, 16)
                    rows_v[b, e, pl.ds(off, 16)] = (
                        rows_v[b, e, pl.ds(off, 16)] * nb)
            return 0
        lax.fori_loop(0, CHUNK // 4, scale_step, 0)

    def run_pass(y_hbm, p):
        def zero_row(e, _):
            for dd in range(D // 16):
                rows_v[0, e, pl.ds(dd * 16, 16)] = jnp.zeros(
                    (16,), jnp.float32)
            return 0
        lax.fori_loop(0, CHUNK, zero_row, 0)

        def zero_sh(i, _):
            pltpu.sync_copy(rows_v.at[0],
                            agg_sh.at[pl.ds(sid * nsl + i * CHUNK, CHUNK)])
            return 0
        lax.fori_loop(0, nsl // CHUNK, zero_sh, 0)
        plsc.subcore_barrier()

        def body(j, b, b2):
            @pl.when(j >= 2)
            def _():
                wait_scatter(j - 2, b2)

            @pl.when(j + 2 < NCHUNK)
            def _():
                fire_gather(y_hbm, j + 2, b2)

            wait_gather(y_hbm, j, b)
            scale(j, b)
            fire_scatter(j, b)

        fire_gather(y_hbm, 0, 0)
        fire_gather(y_hbm, 1, 1)

        def group(q, _):
            for u in range(NBUF):
                j = q * NBUF + u
                body(j, u, (u + 2) % NBUF)
            return 0
        ngroup = NCHUNK // NBUF
        lax.fori_loop(0, ngroup, group, 0)
        for u in range(NCHUNK % NBUF):
            j = ngroup * NBUF + u
            body(j, j % NBUF, (j + 2) % NBUF)

        wait_scatter(NCHUNK - 2, (NCHUNK - 2) % NBUF)
        wait_scatter(NCHUNK - 1, (NCHUNK - 1) % NBUF)

        plsc.subcore_barrier()
        pltpu.sync_copy(agg_sh.at[pl.ds(sid * nsl, nsl)],
                        out_hbm.at[p, cid, pl.ds(sid * nsl, nsl)])

    for p, y_hbm in enumerate(ys):
        run_pass(y_hbm, p)


def _norm_body(roww_hbm, colw_hbm, ww_hbm, norm_hbm,
               row_v, col_v, w_v, deg_v, dis_v, nrm_v,
               deg_sh, dis_sh):
    cid = lax.axis_index("c")
    sid = lax.axis_index("s")
    wid = sid * NC + cid
    nsl = N_PAD // NS

    def zero_dis(i, _):
        off = pl.multiple_of(i * 16, 16)
        dis_v[pl.ds(off, 16)] = jnp.zeros((16,), jnp.float32)
        return 0
    lax.fori_loop(0, nsl // 16, zero_dis, 0)
    pltpu.sync_copy(dis_v, deg_sh.at[pl.ds(sid * nsl, nsl)])
    plsc.subcore_barrier()

    for c2 in range(2):
        slot = sid * 2 + c2
        pltpu.sync_copy(roww_hbm.at[slot], row_v)
        pltpu.sync_copy(ww_hbm.at[slot], w_v)

        def deg_step(j, _):
            pltpu.sync_copy(w_v.at[j], deg_sh.at[row_v.at[j]], add=True)
            return 0
        lax.fori_loop(0, NCHUNK, deg_step, 0)
    plsc.subcore_barrier()

    pltpu.sync_copy(deg_sh.at[pl.ds(sid * nsl, nsl)], dis_v)

    def dis_step(i, _):
        off = pl.multiple_of(i * 16, 16)
        acc = dis_v[pl.ds(off, 16)]
        r = _rsqrt16(acc)
        r = jnp.where(acc > 0.0, r, 0.0)
        dis_v[pl.ds(off, 16)] = r
        return 0
    lax.fori_loop(0, nsl // 16, dis_step, 0)

    pltpu.sync_copy(dis_v, dis_sh.at[pl.ds(sid * nsl, nsl)])
    plsc.subcore_barrier()

    pltpu.sync_copy(dis_sh, deg_v)
    pltpu.sync_copy(roww_hbm.at[wid], row_v)
    pltpu.sync_copy(colw_hbm.at[wid], col_v)
    pltpu.sync_copy(ww_hbm.at[wid], w_v)

    def nrm_step(j, _):
        for dd in range(CHUNK // 16):
            off = pl.multiple_of(dd * 16, 16)
            r = row_v[j, pl.ds(off, 16)]
            c = col_v[j, pl.ds(off, 16)]
            wv = w_v[j, pl.ds(off, 16)]
            dr = plsc.load_gather(deg_v, [r])
            dc = plsc.load_gather(deg_v, [c])
            nrm_v[j, pl.ds(off, 16)] = -(dr * wv * dc)
        return 0
    lax.fori_loop(0, NCHUNK, nrm_step, 0)

    pltpu.sync_copy(nrm_v, norm_hbm.at[wid])


def _norm_call(roww, colw, ww):
    kfn = pl.kernel(
        _norm_body,
        out_type=jax.ShapeDtypeStruct((NW, NCHUNK, CHUNK), jnp.float32),
        mesh=_SC_MESH,
        compiler_params=_SC_PARAMS,
        scratch_types=[
            pltpu.VMEM((NCHUNK, CHUNK), jnp.int32),
            pltpu.VMEM((NCHUNK, CHUNK), jnp.int32),
            pltpu.VMEM((NCHUNK, CHUNK), jnp.float32),
            pltpu.VMEM((N_PAD,), jnp.float32),
            pltpu.VMEM((N_PAD // NS,), jnp.float32),
            pltpu.VMEM((NCHUNK, CHUNK), jnp.float32),
            pltpu.VMEM_SHARED((N_PAD,), jnp.float32),
            pltpu.VMEM_SHARED((N_PAD,), jnp.float32),
        ],
    )
    return kfn(roww, colw, ww)


def _agg_body(npass, *refs):
    ys = refs[:npass]
    (row_hbm, col_hbm, nrm_hbm, out_hbm,
     row_v, col_v, nrm_v, rows_v, agg_sh, gsems, ssems) = refs[npass:]
    cid = lax.axis_index("c")
    sid = lax.axis_index("s")
    wid = sid * NC + cid

    pltpu.sync_copy(row_hbm.at[wid], row_v)
    pltpu.sync_copy(col_hbm.at[wid], col_v)
    pltpu.sync_copy(nrm_hbm.at[wid], nrm_v)

    _agg_passes(ys, out_hbm, row_v, col_v, nrm_v, rows_v,
                agg_sh, gsems, ssems, sid, cid)


def _agg_call(ys, row2, col2, nrm2):
    import functools
    npass = len(ys)
    kfn = pl.kernel(
        functools.partial(_agg_body, npass),
        out_type=jax.ShapeDtypeStruct((npass, NC, N_PAD, D), jnp.float32),
        mesh=_SC_MESH,
        compiler_params=_SC_PARAMS,
        scratch_types=[
            pltpu.VMEM((NCHUNK, CHUNK), jnp.int32),
            pltpu.VMEM((NCHUNK, CHUNK), jnp.int32),
            pltpu.VMEM((NCHUNK, CHUNK), jnp.float32),
            pltpu.VMEM((NBUF, CHUNK, D), jnp.float32),
            pltpu.VMEM_SHARED((N_PAD, D), jnp.float32),
            [pltpu.SemaphoreType.DMA] * NBUF,
            [pltpu.SemaphoreType.DMA] * NBUF,
        ],
    )
    return kfn(*ys, row2, col2, nrm2)


def kernel(features, edge_index, edge_weight, W0_1, W1_1, b1, W0_2, W1_2, b2):
    row = edge_index[0]
    col = edge_index[1]
    row2 = row.reshape(NW, NCHUNK, CHUNK)
    col2 = col.reshape(NW, NCHUNK, CHUNK)

    nrm2 = _norm_call(row2, col2, edge_weight.reshape(NW, NCHUNK, CHUNK))

    wcat1 = jnp.concatenate([W0_1, W1_1], axis=0)
    xw0, y1a, y1b = _mm1(features, wcat1)

    pp = _agg_call([y1a, y1b], row2, col2, nrm2)

    wcat2 = jnp.concatenate([W0_2, W1_2], axis=0)
    hw0, y2 = _combine1(xw0, pp[0, 0, :N], pp[0, 1, :N],
                        pp[1, 0, :N], pp[1, 1, :N],
                        b1.reshape(1, NHID), wcat2)

    qq = _agg_call([y2], row2, col2, nrm2)
    q = qq[0]

    return _final(hw0, q[0, :N], q[1, :N], b2.reshape(1, NCLASS))
